# Initial kernel scaffold; baseline (speedup 1.0000x reference)
import jax
import jax.numpy as jnp
from jax.experimental import pallas as pl


def kernel(triplets, corrupted_triplets, entity_emb, relation_emb):
    raise NotImplementedError("write your pallas kernel here")


# speedup vs baseline: 1.2492x; 1.2492x over previous
import functools

import jax
import jax.numpy as jnp
from jax import lax
from jax.experimental import pallas as pl
from jax.experimental.pallas import tpu as pltpu
from jax.experimental.pallas import tpu_sc as plsc

B = 16384
DIM = 64
MARGIN = 1.0
EPS = 1e-12


def _sc_gather(entity_emb, relation_emb, heads, rels, tails, ctails):
    info = plsc.get_sparse_core_info()
    nc, ns = info.num_cores, info.num_subcores
    nw = nc * ns
    bpw = B // nw
    mesh = plsc.VectorSubcoreMesh(core_axis_name="c", subcore_axis_name="s")

    @functools.partial(
        pl.kernel,
        mesh=mesh,
        out_type=[jax.ShapeDtypeStruct((B, DIM), jnp.float32)] * 4,
        scratch_types=[
            pltpu.VMEM((bpw,), jnp.int32),
            pltpu.VMEM((bpw, DIM), jnp.float32),
            pltpu.SemaphoreType.DMA,
        ],
        compiler_params=pltpu.CompilerParams(use_tc_tiling_on_sc=False),
    )
    def gather_kernel(ent_hbm, rel_hbm, h_hbm, r_hbm, t_hbm, c_hbm,
                      h_out, t_out, c_out, r_out, idx_v, rows_v, sem):
        wid = lax.axis_index("s") * nc + lax.axis_index("c")
        base = wid * bpw
        for src_tab, src_idx, dst in (
            (ent_hbm, h_hbm, h_out),
            (ent_hbm, t_hbm, t_out),
            (ent_hbm, c_hbm, c_out),
            (rel_hbm, r_hbm, r_out),
        ):
            pltpu.sync_copy(src_idx.at[pl.ds(base, bpw)], idx_v)
            pltpu.async_copy(src_tab.at[idx_v], rows_v, sem).wait()
            pltpu.sync_copy(rows_v, dst.at[pl.ds(base, bpw)])

    return gather_kernel(entity_emb, relation_emb, heads, rels, tails, ctails)


def _tc_loss(h_rows, t_rows, c_rows, r_rows):
    blk = 2048
    nblk = B // blk

    def body(h_ref, t_ref, c_ref, r_ref, o_ref, acc_ref):
        step = pl.program_id(0)

        @pl.when(step == 0)
        def _init():
            acc_ref[0, 0] = 0.0

        h = h_ref[...]
        t = t_ref[...]
        c = c_ref[...]
        r = r_ref[...]
        h = h / jnp.sqrt(jnp.sum(h * h, axis=1, keepdims=True))
        t = t / jnp.sqrt(jnp.sum(t * t, axis=1, keepdims=True))
        c = c / jnp.sqrt(jnp.sum(c * c, axis=1, keepdims=True))
        dp = h + r - t
        dn = h + r - c
        pos_dist = jnp.sqrt(jnp.sum(dp * dp, axis=1) + EPS)
        neg_dist = jnp.sqrt(jnp.sum(dn * dn, axis=1) + EPS)
        terms = jnp.maximum(0.0, MARGIN + pos_dist - neg_dist)
        acc_ref[0, 0] += jnp.sum(terms)

        @pl.when(step == nblk - 1)
        def _fini():
            o_ref[0, 0] = acc_ref[0, 0] * (1.0 / B)

    row_spec = pl.BlockSpec((blk, DIM), lambda i: (i, 0))
    out = pl.pallas_call(
        body,
        grid=(nblk,),
        in_specs=[row_spec] * 4,
        out_specs=pl.BlockSpec(memory_space=pltpu.SMEM),
        out_shape=jax.ShapeDtypeStruct((1, 1), jnp.float32),
        scratch_shapes=[pltpu.SMEM((1, 1), jnp.float32)],
    )(h_rows, t_rows, c_rows, r_rows)
    return out[0, 0]


def kernel(triplets, corrupted_triplets, entity_emb, relation_emb):
    heads = triplets[:, 0].astype(jnp.int32)
    rels = triplets[:, 1].astype(jnp.int32)
    tails = triplets[:, 2].astype(jnp.int32)
    ctails = corrupted_triplets[:, 2].astype(jnp.int32)
    h_rows, t_rows, c_rows, r_rows = _sc_gather(
        entity_emb, relation_emb, heads, rels, tails, ctails)
    return _tc_loss(h_rows, t_rows, c_rows, r_rows)


# speedup vs baseline: 1.7790x; 1.4241x over previous
import functools

import jax
import jax.numpy as jnp
from jax import lax
from jax.experimental import pallas as pl
from jax.experimental.pallas import tpu as pltpu
from jax.experimental.pallas import tpu_sc as plsc

B = 16384
DIM = 64
N_ENT = 1000000
MARGIN = 1.0
EPS = 1e-12

W = 512
NFULL = N_ENT // W
TAILW = N_ENT - NFULL * W
TAIL_ID = NFULL
NI = 3 * B
PIECE = 4096
NPIECE = NI // PIECE
CAP = 2048
BPW = B // 32


def _sc_gather(tab_t, rel_t, all_idx, rels):
    info = plsc.get_sparse_core_info()
    nc = info.num_cores
    mesh = plsc.VectorSubcoreMesh(core_axis_name="c", subcore_axis_name="s")

    @functools.partial(
        pl.kernel,
        mesh=mesh,
        out_type=[
            jax.ShapeDtypeStruct((NI + 16, 128), jnp.float32),
            jax.ShapeDtypeStruct((B, 128), jnp.float32),
        ],
        scratch_types=[
            pltpu.VMEM((DIM, W), jnp.float32),
            pltpu.VMEM((DIM, TAILW), jnp.float32),
            pltpu.VMEM((DIM, 500), jnp.float32),
            pltpu.VMEM((PIECE,), jnp.int32),
            pltpu.VMEM((BPW,), jnp.int32),
            pltpu.VMEM((CAP,), jnp.int32),
            pltpu.VMEM((CAP,), jnp.int32),
            pltpu.VMEM((CAP,), jnp.int32),
            pltpu.VMEM((CAP,), jnp.int32),
            pltpu.VMEM((16, 128), jnp.float32),
            pltpu.SemaphoreType.DMA,
        ],
        compiler_params=pltpu.CompilerParams(needs_layout_passes=False),
    )
    def k(tab_hbm, rel_hbm, idx_hbm, rels_hbm, out_hbm, rel_out_hbm,
          buf_v, tail_v, relbuf_v, piece_v, rels_v, mli_v, mlo_v,
          wli_v, wlo_v, stag_v, sem):
        wid = lax.axis_index("s") * nc + lax.axis_index("c")
        iota16 = lax.iota(jnp.int32, 16)

        pltpu.sync_copy(rel_hbm, relbuf_v)
        pltpu.sync_copy(rels_hbm.at[pl.ds(wid * BPW, BPW)], rels_v)

        def rel_grp(g, _):
            rvec = rels_v[pl.ds(g * 16, 16)]
            for c0 in range(DIM):
                cvec = jnp.full((16,), c0, jnp.int32)
                vals = plsc.load_gather(relbuf_v, [cvec, rvec])
                plsc.store_scatter(stag_v, [iota16, cvec], vals)
            pltpu.sync_copy(
                stag_v, rel_out_hbm.at[pl.ds(wid * BPW + g * 16, 16)])
            return 0

        lax.fori_loop(0, BPW // 16, rel_grp, 0)

        def scan(rnd):
            lo = rnd * CAP

            def scan_piece(p, cnt):
                pltpu.sync_copy(idx_hbm.at[pl.ds(p * PIECE, PIECE)], piece_v)

                def scan_vec(i, cnt):
                    vec = piece_v[pl.ds(i * 16, 16)]
                    mine = ((vec // W) & 31) == wid
                    mi = mine.astype(jnp.int32)
                    rank = cnt + lax.cumsum(mi, axis=0) - mi
                    keep = mine & (rank >= lo) & (rank < lo + CAP)
                    pos = jnp.clip(rank - lo, 0, CAP - 1)
                    org = p * PIECE + i * 16 + iota16
                    plsc.store_scatter(mli_v, [pos], vec, mask=keep)
                    plsc.store_scatter(mlo_v, [pos], org, mask=keep)
                    return cnt + jnp.sum(mi)

                return lax.fori_loop(0, PIECE // 16, scan_vec, cnt)

            return lax.fori_loop(0, NPIECE, scan_piece, jnp.int32(0))

        def compact(ch, mr):
            def cvec_it(ki, c2):
                vec = mli_v[pl.ds(ki * 16, 16)]
                org = mlo_v[pl.ds(ki * 16, 16)]
                m = ((ki * 16 + iota16) < mr) & ((vec // W) == ch)
                mi = m.astype(jnp.int32)
                pos = c2 + lax.cumsum(mi, axis=0) - mi
                plsc.store_scatter(wli_v, [pos], vec, mask=m)
                plsc.store_scatter(wlo_v, [pos], org, mask=m)
                return c2 + jnp.sum(mi)

            return lax.fori_loop(0, (mr + 15) // 16, cvec_it, jnp.int32(0))

        def extract(src_v, width, base, c2):
            def grp(g, _):
                lanes0 = wli_v[pl.ds(g * 16, 16)]
                valid = (g * 16 + iota16) < c2
                lanes = jnp.clip(jnp.where(valid, lanes0 - base, 0),
                                 0, width - 1)
                for c0 in range(DIM):
                    cvec = jnp.full((16,), c0, jnp.int32)
                    vals = plsc.load_gather(src_v, [cvec, lanes])
                    plsc.store_scatter(stag_v, [iota16, cvec], vals)
                org = jnp.where(valid, wlo_v[pl.ds(g * 16, 16)], NI + iota16)
                pltpu.async_copy(stag_v, out_hbm.at[org], sem).wait()
                return 0

            lax.fori_loop(0, (c2 + 15) // 16, grp, 0)

        def round_body(carry):
            rnd, _ = carry
            cnt = scan(rnd)
            mr = jnp.clip(cnt - rnd * CAP, 0, CAP)

            def do_chunk(j, _):
                ch = wid + 32 * j

                @pl.when(ch < NFULL)
                def _full():
                    pltpu.sync_copy(tab_hbm.at[:, pl.ds(ch * W, W)], buf_v)
                    extract(buf_v, W, ch * W, compact(ch, mr))

                @pl.when(ch == TAIL_ID)
                def _tail():
                    pltpu.sync_copy(
                        tab_hbm.at[:, pl.ds(NFULL * W, TAILW)], tail_v)
                    extract(tail_v, TAILW, NFULL * W, compact(ch, mr))

                return 0

            lax.fori_loop(0, (TAIL_ID - wid) // 32 + 1, do_chunk, 0)
            return rnd + 1, cnt

        lax.while_loop(lambda c: c[0] * CAP < c[1], round_body,
                       (jnp.int32(0), jnp.int32(1)))

    return k(tab_t, rel_t, all_idx, rels)


def _tc_loss(h_rows, t_rows, c_rows, r_rows):
    blk = 2048
    nblk = B // blk

    def body(h_ref, t_ref, c_ref, r_ref, o_ref, acc_ref):
        step = pl.program_id(0)

        @pl.when(step == 0)
        def _init():
            acc_ref[0, 0] = 0.0

        h = h_ref[...][:, :DIM]
        t = t_ref[...][:, :DIM]
        c = c_ref[...][:, :DIM]
        r = r_ref[...][:, :DIM]
        h = h / jnp.sqrt(jnp.sum(h * h, axis=1, keepdims=True))
        t = t / jnp.sqrt(jnp.sum(t * t, axis=1, keepdims=True))
        c = c / jnp.sqrt(jnp.sum(c * c, axis=1, keepdims=True))
        dp = h + r - t
        dn = h + r - c
        pos_dist = jnp.sqrt(jnp.sum(dp * dp, axis=1) + EPS)
        neg_dist = jnp.sqrt(jnp.sum(dn * dn, axis=1) + EPS)
        terms = jnp.maximum(0.0, MARGIN + pos_dist - neg_dist)
        acc_ref[0, 0] += jnp.sum(terms)

        @pl.when(step == nblk - 1)
        def _fini():
            o_ref[0, 0] = acc_ref[0, 0] * (1.0 / B)

    row_spec = pl.BlockSpec((blk, 128), lambda i: (i, 0))
    out = pl.pallas_call(
        body,
        grid=(nblk,),
        in_specs=[row_spec] * 4,
        out_specs=pl.BlockSpec(memory_space=pltpu.SMEM),
        out_shape=jax.ShapeDtypeStruct((1, 1), jnp.float32),
        scratch_shapes=[pltpu.SMEM((1, 1), jnp.float32)],
    )(h_rows, t_rows, c_rows, r_rows)
    return out[0, 0]


def kernel(triplets, corrupted_triplets, entity_emb, relation_emb):
    heads = triplets[:, 0].astype(jnp.int32)
    rels = triplets[:, 1].astype(jnp.int32)
    tails = triplets[:, 2].astype(jnp.int32)
    ctails = corrupted_triplets[:, 2].astype(jnp.int32)
    all_idx = jnp.concatenate([heads, tails, ctails])
    ent_rows, rel_rows = _sc_gather(
        entity_emb.T, relation_emb.T, all_idx, rels)
    return _tc_loss(ent_rows[:B], ent_rows[B:2 * B], ent_rows[2 * B:3 * B],
                    rel_rows)


# speedup vs baseline: 1.8519x; 1.0410x over previous
import functools

import jax
import jax.numpy as jnp
from jax import lax
from jax.experimental import pallas as pl
from jax.experimental.pallas import tpu as pltpu
from jax.experimental.pallas import tpu_sc as plsc

B = 16384
DIM = 64
N_ENT = 1000000
MARGIN = 1.0
EPS = 1e-12

W = 512
WSH = 9
NFULL = N_ENT // W
TAILW = N_ENT - NFULL * W
TAIL_ID = NFULL
NI = 3 * B
PIECE = 4096
NPIECE = NI // PIECE
CAP = 2048
BPW = B // 32


def _sc_gather(tab_t, rel_t, all_idx, rels):
    info = plsc.get_sparse_core_info()
    nc = info.num_cores
    mesh = plsc.VectorSubcoreMesh(core_axis_name="c", subcore_axis_name="s")

    @functools.partial(
        pl.kernel,
        mesh=mesh,
        out_type=[
            jax.ShapeDtypeStruct((NI + 16, 128), jnp.float32),
            jax.ShapeDtypeStruct((B, 128), jnp.float32),
        ],
        scratch_types=[
            pltpu.VMEM((DIM, W), jnp.float32),
            pltpu.VMEM((DIM, TAILW), jnp.float32),
            pltpu.VMEM((DIM, 500), jnp.float32),
            pltpu.VMEM((PIECE,), jnp.int32),
            pltpu.VMEM((BPW,), jnp.int32),
            pltpu.VMEM((CAP,), jnp.int32),
            pltpu.VMEM((CAP,), jnp.int32),
            pltpu.VMEM((CAP,), jnp.int32),
            pltpu.VMEM((CAP,), jnp.int32),
            pltpu.VMEM((16, 128), jnp.float32),
            pltpu.VMEM((16, 128), jnp.float32),
            pltpu.SemaphoreType.DMA,
            pltpu.SemaphoreType.DMA,
        ],
        compiler_params=pltpu.CompilerParams(needs_layout_passes=False),
    )
    def k(tab_hbm, rel_hbm, idx_hbm, rels_hbm, out_hbm, rel_out_hbm,
          buf_v, tail_v, relbuf_v, piece_v, rels_v, mli_v, mlo_v,
          wli_v, wlo_v, stag0_v, stag1_v, sem0, sem1):
        wid = lax.axis_index("s") * nc + lax.axis_index("c")
        iota16 = lax.iota(jnp.int32, 16)
        stags = (stag0_v, stag1_v)
        sems = (sem0, sem1)

        def ring_wait(b):
            pltpu.make_async_copy(
                stags[b], out_hbm.at[pl.ds(NI, 16)], sems[b]).wait()

        pltpu.sync_copy(rel_hbm, relbuf_v)
        pltpu.sync_copy(rels_hbm.at[pl.ds(wid * BPW, BPW)], rels_v)

        def rel_grp(g, _):
            for b in (0, 1):
                @pl.when((g & 1) == b)
                def _b():
                    @pl.when(g >= 2)
                    def _w():
                        ring_wait(b)

                    rvec = rels_v[pl.ds(g * 16, 16)]
                    for c0 in range(DIM):
                        cvec = jnp.full((16,), c0, jnp.int32)
                        vals = plsc.load_gather(relbuf_v, [cvec, rvec])
                        plsc.store_scatter(stags[b], [iota16, cvec], vals)
                    pltpu.async_copy(
                        stags[b],
                        rel_out_hbm.at[pl.ds(wid * BPW + g * 16, 16)],
                        sems[b])

            return 0

        lax.fori_loop(0, BPW // 16, rel_grp, 0)
        ring_wait(0)
        ring_wait(1)

        def scan(rnd):
            lo = rnd * CAP

            def scan_piece(p, cnt):
                pltpu.sync_copy(idx_hbm.at[pl.ds(p * PIECE, PIECE)], piece_v)

                def scan_vec(i, cnt):
                    vec = piece_v[pl.ds(i * 16, 16)]
                    mine = ((vec >> WSH) & 31) == wid
                    mi = mine.astype(jnp.int32)
                    rank = cnt + lax.cumsum(mi, axis=0) - mi
                    keep = mine & (rank >= lo) & (rank < lo + CAP)
                    pos = jnp.clip(rank - lo, 0, CAP - 1)
                    org = p * PIECE + i * 16 + iota16
                    plsc.store_scatter(mli_v, [pos], vec, mask=keep)
                    plsc.store_scatter(mlo_v, [pos], org, mask=keep)
                    return cnt + plsc.all_reduce_population_count(mine)

                return lax.fori_loop(0, PIECE // 16, scan_vec, cnt)

            cnt = lax.fori_loop(0, NPIECE, scan_piece,
                                jnp.zeros((16,), jnp.int32))
            return jnp.max(cnt)

        def compact(ch, mr):
            def cvec_it(ki, c2):
                vec = mli_v[pl.ds(ki * 16, 16)]
                org = mlo_v[pl.ds(ki * 16, 16)]
                m = ((ki * 16 + iota16) < mr) & ((vec >> WSH) == ch)
                mi = m.astype(jnp.int32)
                pos = c2 + lax.cumsum(mi, axis=0) - mi
                plsc.store_scatter(wli_v, [pos], vec, mask=m)
                plsc.store_scatter(wlo_v, [pos], org, mask=m)
                return c2 + plsc.all_reduce_population_count(m)

            c2 = lax.fori_loop(0, (mr + 15) // 16, cvec_it,
                               jnp.zeros((16,), jnp.int32))
            return jnp.max(c2)

        def extract(src_v, width, base, c2):
            def grp(g, _):
                for b in (0, 1):
                    @pl.when((g & 1) == b)
                    def _b():
                        @pl.when(g >= 2)
                        def _w():
                            ring_wait(b)

                        lanes0 = wli_v[pl.ds(g * 16, 16)]
                        valid = (g * 16 + iota16) < c2
                        lanes = jnp.clip(jnp.where(valid, lanes0 - base, 0),
                                         0, width - 1)
                        for c0 in range(DIM):
                            cvec = jnp.full((16,), c0, jnp.int32)
                            vals = plsc.load_gather(src_v, [cvec, lanes])
                            plsc.store_scatter(stags[b], [iota16, cvec], vals)
                        org = jnp.where(valid, wlo_v[pl.ds(g * 16, 16)],
                                        NI + iota16)
                        pltpu.async_copy(stags[b], out_hbm.at[org], sems[b])

                return 0

            ngrp = (c2 + 15) // 16
            lax.fori_loop(0, ngrp, grp, 0)

            @pl.when(ngrp >= 1)
            def _d0():
                ring_wait(0)

            @pl.when(ngrp >= 2)
            def _d1():
                ring_wait(1)

        def round_body(carry):
            rnd, _ = carry
            cnt = scan(rnd)
            mr = jnp.clip(cnt - rnd * CAP, 0, CAP)

            def do_chunk(j, _):
                ch = wid + 32 * j

                @pl.when(ch < NFULL)
                def _full():
                    pltpu.sync_copy(tab_hbm.at[:, pl.ds(ch * W, W)], buf_v)
                    extract(buf_v, W, ch * W, compact(ch, mr))

                @pl.when(ch == TAIL_ID)
                def _tail():
                    pltpu.sync_copy(
                        tab_hbm.at[:, pl.ds(NFULL * W, TAILW)], tail_v)
                    extract(tail_v, TAILW, NFULL * W, compact(ch, mr))

                return 0

            lax.fori_loop(0, (TAIL_ID - wid) // 32 + 1, do_chunk, 0)
            return rnd + 1, cnt

        lax.while_loop(lambda c: c[0] * CAP < c[1], round_body,
                       (jnp.int32(0), jnp.int32(1)))

    return k(tab_t, rel_t, all_idx, rels)


def _tc_loss(h_rows, t_rows, c_rows, r_rows):
    blk = 2048
    nblk = B // blk

    def body(h_ref, t_ref, c_ref, r_ref, o_ref, acc_ref):
        step = pl.program_id(0)

        @pl.when(step == 0)
        def _init():
            acc_ref[0, 0] = 0.0

        h = h_ref[...][:, :DIM]
        t = t_ref[...][:, :DIM]
        c = c_ref[...][:, :DIM]
        r = r_ref[...][:, :DIM]
        h = h / jnp.sqrt(jnp.sum(h * h, axis=1, keepdims=True))
        t = t / jnp.sqrt(jnp.sum(t * t, axis=1, keepdims=True))
        c = c / jnp.sqrt(jnp.sum(c * c, axis=1, keepdims=True))
        dp = h + r - t
        dn = h + r - c
        pos_dist = jnp.sqrt(jnp.sum(dp * dp, axis=1) + EPS)
        neg_dist = jnp.sqrt(jnp.sum(dn * dn, axis=1) + EPS)
        terms = jnp.maximum(0.0, MARGIN + pos_dist - neg_dist)
        acc_ref[0, 0] += jnp.sum(terms)

        @pl.when(step == nblk - 1)
        def _fini():
            o_ref[0, 0] = acc_ref[0, 0] * (1.0 / B)

    row_spec = pl.BlockSpec((blk, 128), lambda i: (i, 0))
    out = pl.pallas_call(
        body,
        grid=(nblk,),
        in_specs=[row_spec] * 4,
        out_specs=pl.BlockSpec(memory_space=pltpu.SMEM),
        out_shape=jax.ShapeDtypeStruct((1, 1), jnp.float32),
        scratch_shapes=[pltpu.SMEM((1, 1), jnp.float32)],
    )(h_rows, t_rows, c_rows, r_rows)
    return out[0, 0]


def kernel(triplets, corrupted_triplets, entity_emb, relation_emb):
    heads = triplets[:, 0].astype(jnp.int32)
    rels = triplets[:, 1].astype(jnp.int32)
    tails = triplets[:, 2].astype(jnp.int32)
    ctails = corrupted_triplets[:, 2].astype(jnp.int32)
    all_idx = jnp.concatenate([heads, tails, ctails])
    ent_rows, rel_rows = _sc_gather(
        entity_emb.T, relation_emb.T, all_idx, rels)
    return _tc_loss(ent_rows[:B], ent_rows[B:2 * B], ent_rows[2 * B:3 * B],
                    rel_rows)


# speedup vs baseline: 2.1045x; 1.1364x over previous
import functools

import jax
import jax.numpy as jnp
from jax import lax
from jax.experimental import pallas as pl
from jax.experimental.pallas import tpu as pltpu
from jax.experimental.pallas import tpu_sc as plsc

B = 16384
DIM = 64
N_ENT = 1000000
MARGIN = 1.0
EPS = 1e-12

W = 512
WSH = 9
NFULL = N_ENT // W
TAILW = N_ENT - NFULL * W
TAIL_ID = NFULL
NI = 3 * B
PIECE = 4096
NPIECE = NI // PIECE
CAP = 2048
BPW = B // 32


def _sc_gather(tab_t, rel_t, all_idx, rels):
    info = plsc.get_sparse_core_info()
    nc = info.num_cores
    mesh = plsc.VectorSubcoreMesh(core_axis_name="c", subcore_axis_name="s")

    @functools.partial(
        pl.kernel,
        mesh=mesh,
        out_type=[
            jax.ShapeDtypeStruct((NI + 16, 128), jnp.float32),
            jax.ShapeDtypeStruct((B, 128), jnp.float32),
        ],
        scratch_types=[
            pltpu.VMEM((DIM, W), jnp.float32),
            pltpu.VMEM((DIM, W), jnp.float32),
            pltpu.VMEM((PIECE,), jnp.int32),
            pltpu.SemaphoreType.DMA,
            pltpu.SemaphoreType.DMA,
            pltpu.SemaphoreType.DMA,
            pltpu.SemaphoreType.DMA,
            pltpu.VMEM((DIM, TAILW), jnp.float32),
            pltpu.VMEM((DIM, 500), jnp.float32),
            pltpu.VMEM((PIECE,), jnp.int32),
            pltpu.VMEM((BPW,), jnp.int32),
            pltpu.VMEM((CAP,), jnp.int32),
            pltpu.VMEM((CAP,), jnp.int32),
            pltpu.VMEM((CAP,), jnp.int32),
            pltpu.VMEM((CAP,), jnp.int32),
            pltpu.VMEM((16, 128), jnp.float32),
            pltpu.VMEM((16, 128), jnp.float32),
            pltpu.SemaphoreType.DMA,
            pltpu.SemaphoreType.DMA,
        ],
        compiler_params=pltpu.CompilerParams(needs_layout_passes=False),
    )
    def k(tab_hbm, rel_hbm, idx_hbm, rels_hbm, out_hbm, rel_out_hbm,
          bufa_v, bufb_v, pieceb_v, csem0, csem1, psem0, psem1,
          tail_v, relbuf_v, piece_v, rels_v, mli_v, mlo_v,
          wli_v, wlo_v, stag0_v, stag1_v, sem0, sem1):
        wid = lax.axis_index("s") * nc + lax.axis_index("c")
        iota16 = lax.iota(jnp.int32, 16)
        stags = (stag0_v, stag1_v)
        sems = (sem0, sem1)
        bufs = (bufa_v, bufb_v)
        csems = (csem0, csem1)
        pieces = (piece_v, pieceb_v)
        psems = (psem0, psem1)

        def ring_wait(b):
            pltpu.make_async_copy(
                stags[b], out_hbm.at[pl.ds(NI, 16)], sems[b]).wait()

        pltpu.sync_copy(rel_hbm, relbuf_v)
        pltpu.sync_copy(rels_hbm.at[pl.ds(wid * BPW, BPW)], rels_v)

        def rel_grp(g, _):
            for b in (0, 1):
                @pl.when((g & 1) == b)
                def _b():
                    @pl.when(g >= 2)
                    def _w():
                        ring_wait(b)

                    rvec = rels_v[pl.ds(g * 16, 16)]
                    for c0 in range(DIM):
                        cvec = jnp.full((16,), c0, jnp.int32)
                        vals = plsc.load_gather(relbuf_v, [cvec, rvec])
                        plsc.store_scatter(stags[b], [iota16, cvec], vals)
                    pltpu.async_copy(
                        stags[b],
                        rel_out_hbm.at[pl.ds(wid * BPW + g * 16, 16)],
                        sems[b])

            return 0

        lax.fori_loop(0, BPW // 16, rel_grp, 0)
        ring_wait(0)
        ring_wait(1)

        def scan(rnd):
            lo = rnd * CAP

            def piece_issue(p, b):
                pltpu.async_copy(
                    idx_hbm.at[pl.ds(p * PIECE, PIECE)], pieces[b], psems[b])

            def piece_wait(b):
                pltpu.make_async_copy(
                    idx_hbm.at[pl.ds(0, PIECE)], pieces[b], psems[b]).wait()

            piece_issue(0, 0)
            piece_issue(1, 1)

            def scan_piece(p, cnt):
                par0 = (p & 1) == 0

                for b in (0, 1):
                    @pl.when((p & 1) == b)
                    def _w():
                        piece_wait(b)

                def scan_vec(i, cnt):
                    v0 = pieces[0][pl.ds(i * 16, 16)]
                    v1 = pieces[1][pl.ds(i * 16, 16)]
                    vec = jnp.where(par0, v0, v1)
                    mine = ((vec >> WSH) & 31) == wid
                    mi = mine.astype(jnp.int32)
                    rank = cnt + lax.cumsum(mi, axis=0) - mi
                    keep = mine & (rank >= lo) & (rank < lo + CAP)
                    pos = jnp.clip(rank - lo, 0, CAP - 1)
                    org = p * PIECE + i * 16 + iota16
                    plsc.store_scatter(mli_v, [pos], vec, mask=keep)
                    plsc.store_scatter(mlo_v, [pos], org, mask=keep)
                    return cnt + plsc.all_reduce_population_count(mine)

                cnt = lax.fori_loop(0, PIECE // 16, scan_vec, cnt)

                for b in (0, 1):
                    @pl.when(((p & 1) == b) & (p + 2 < NPIECE))
                    def _i():
                        piece_issue(p + 2, b)

                return cnt

            cnt = lax.fori_loop(0, NPIECE, scan_piece,
                                jnp.zeros((16,), jnp.int32))
            return jnp.max(cnt)

        def compact(ch, mr):
            def cvec_it(ki, c2):
                vec = mli_v[pl.ds(ki * 16, 16)]
                org = mlo_v[pl.ds(ki * 16, 16)]
                m = ((ki * 16 + iota16) < mr) & ((vec >> WSH) == ch)
                mi = m.astype(jnp.int32)
                pos = c2 + lax.cumsum(mi, axis=0) - mi
                plsc.store_scatter(wli_v, [pos], vec, mask=m)
                plsc.store_scatter(wlo_v, [pos], org, mask=m)
                return c2 + plsc.all_reduce_population_count(m)

            c2 = lax.fori_loop(0, (mr + 15) // 16, cvec_it,
                               jnp.zeros((16,), jnp.int32))
            return jnp.max(c2)

        def extract(src_v, width, base, c2):
            def grp(g, _):
                for b in (0, 1):
                    @pl.when((g & 1) == b)
                    def _b():
                        @pl.when(g >= 2)
                        def _w():
                            ring_wait(b)

                        lanes0 = wli_v[pl.ds(g * 16, 16)]
                        valid = (g * 16 + iota16) < c2
                        lanes = jnp.clip(jnp.where(valid, lanes0 - base, 0),
                                         0, width - 1)
                        for c0 in range(DIM):
                            cvec = jnp.full((16,), c0, jnp.int32)
                            vals = plsc.load_gather(src_v, [cvec, lanes])
                            plsc.store_scatter(stags[b], [iota16, cvec], vals)
                        org = jnp.where(valid, wlo_v[pl.ds(g * 16, 16)],
                                        NI + iota16)
                        pltpu.async_copy(stags[b], out_hbm.at[org], sems[b])

                return 0

            ngrp = (c2 + 15) // 16
            lax.fori_loop(0, ngrp, grp, 0)

            @pl.when(ngrp >= 1)
            def _d0():
                ring_wait(0)

            @pl.when(ngrp >= 2)
            def _d1():
                ring_wait(1)

        def chunk_issue(j, b):
            ch = wid + 32 * j
            pltpu.async_copy(
                tab_hbm.at[:, pl.ds(ch * W, W)], bufs[b], csems[b])

        def chunk_wait(b):
            pltpu.make_async_copy(
                tab_hbm.at[:, pl.ds(0, W)], bufs[b], csems[b]).wait()

        def round_body(carry):
            rnd, _ = carry
            cnt = scan(rnd)
            mr = jnp.clip(cnt - rnd * CAP, 0, CAP)
            nfm = (NFULL - 1 - wid) // 32 + 1

            chunk_issue(0, 0)

            @pl.when(nfm >= 2)
            def _p1():
                chunk_issue(1, 1)

            def do_chunk(j, _):
                for b in (0, 1):
                    @pl.when((j & 1) == b)
                    def _b():
                        chunk_wait(b)
                        ch = wid + 32 * j
                        extract(bufs[b], W, ch * W, compact(ch, mr))

                        @pl.when(j + 2 < nfm)
                        def _pf():
                            chunk_issue(j + 2, b)

                return 0

            lax.fori_loop(0, nfm, do_chunk, 0)

            @pl.when(wid == (TAIL_ID & 31))
            def _tail():
                pltpu.sync_copy(
                    tab_hbm.at[:, pl.ds(NFULL * W, TAILW)], tail_v)
                extract(tail_v, TAILW, NFULL * W, compact(TAIL_ID, mr))

            return rnd + 1, cnt

        lax.while_loop(lambda c: c[0] * CAP < c[1], round_body,
                       (jnp.int32(0), jnp.int32(1)))

    return k(tab_t, rel_t, all_idx, rels)


def _tc_loss(h_rows, t_rows, c_rows, r_rows):
    blk = 2048
    nblk = B // blk

    def body(h_ref, t_ref, c_ref, r_ref, o_ref, acc_ref):
        step = pl.program_id(0)

        @pl.when(step == 0)
        def _init():
            acc_ref[0, 0] = 0.0

        h = h_ref[...][:, :DIM]
        t = t_ref[...][:, :DIM]
        c = c_ref[...][:, :DIM]
        r = r_ref[...][:, :DIM]
        h = h / jnp.sqrt(jnp.sum(h * h, axis=1, keepdims=True))
        t = t / jnp.sqrt(jnp.sum(t * t, axis=1, keepdims=True))
        c = c / jnp.sqrt(jnp.sum(c * c, axis=1, keepdims=True))
        dp = h + r - t
        dn = h + r - c
        pos_dist = jnp.sqrt(jnp.sum(dp * dp, axis=1) + EPS)
        neg_dist = jnp.sqrt(jnp.sum(dn * dn, axis=1) + EPS)
        terms = jnp.maximum(0.0, MARGIN + pos_dist - neg_dist)
        acc_ref[0, 0] += jnp.sum(terms)

        @pl.when(step == nblk - 1)
        def _fini():
            o_ref[0, 0] = acc_ref[0, 0] * (1.0 / B)

    row_spec = pl.BlockSpec((blk, 128), lambda i: (i, 0))
    out = pl.pallas_call(
        body,
        grid=(nblk,),
        in_specs=[row_spec] * 4,
        out_specs=pl.BlockSpec(memory_space=pltpu.SMEM),
        out_shape=jax.ShapeDtypeStruct((1, 1), jnp.float32),
        scratch_shapes=[pltpu.SMEM((1, 1), jnp.float32)],
    )(h_rows, t_rows, c_rows, r_rows)
    return out[0, 0]


def kernel(triplets, corrupted_triplets, entity_emb, relation_emb):
    heads = triplets[:, 0].astype(jnp.int32)
    rels = triplets[:, 1].astype(jnp.int32)
    tails = triplets[:, 2].astype(jnp.int32)
    ctails = corrupted_triplets[:, 2].astype(jnp.int32)
    all_idx = jnp.concatenate([heads, tails, ctails])
    ent_rows, rel_rows = _sc_gather(
        entity_emb.T, relation_emb.T, all_idx, rels)
    return _tc_loss(ent_rows[:B], ent_rows[B:2 * B], ent_rows[2 * B:3 * B],
                    rel_rows)


# speedup vs baseline: 2.1147x; 1.0048x over previous
import functools

import jax
import jax.numpy as jnp
from jax import lax
from jax.experimental import pallas as pl
from jax.experimental.pallas import tpu as pltpu
from jax.experimental.pallas import tpu_sc as plsc

B = 16384
DIM = 64
N_ENT = 1000000
MARGIN = 1.0
EPS = 1e-12

W = 512
WSH = 9
NFULL = N_ENT // W
TAILW = N_ENT - NFULL * W
TAIL_ID = NFULL
NI = 3 * B
PIECE = 4096
NPIECE = NI // PIECE
CAP = 2048
BPW = B // 32


def _sc_gather(tab_t, rel_t, all_idx, rels):
    info = plsc.get_sparse_core_info()
    nc = info.num_cores
    mesh = plsc.VectorSubcoreMesh(core_axis_name="c", subcore_axis_name="s")

    @functools.partial(
        pl.kernel,
        mesh=mesh,
        out_type=[
            jax.ShapeDtypeStruct((NI + 16, 128), jnp.float32),
            jax.ShapeDtypeStruct((B, 128), jnp.float32),
        ],
        scratch_types=[
            pltpu.VMEM((DIM, W), jnp.float32),
            pltpu.VMEM((DIM, W), jnp.float32),
            pltpu.VMEM((PIECE,), jnp.int32),
            pltpu.SemaphoreType.DMA,
            pltpu.SemaphoreType.DMA,
            pltpu.SemaphoreType.DMA,
            pltpu.SemaphoreType.DMA,
            pltpu.VMEM((DIM, TAILW), jnp.float32),
            pltpu.VMEM((DIM, 500), jnp.float32),
            pltpu.VMEM((PIECE,), jnp.int32),
            pltpu.VMEM((BPW,), jnp.int32),
            pltpu.VMEM((CAP,), jnp.int32),
            pltpu.VMEM((CAP,), jnp.int32),
            pltpu.VMEM((CAP,), jnp.int32),
            pltpu.VMEM((CAP,), jnp.int32),
            pltpu.VMEM((16, 128), jnp.float32),
            pltpu.VMEM((16, 128), jnp.float32),
            pltpu.SemaphoreType.DMA,
            pltpu.SemaphoreType.DMA,
        ],
        compiler_params=pltpu.CompilerParams(needs_layout_passes=False),
    )
    def k(tab_hbm, rel_hbm, idx_hbm, rels_hbm, out_hbm, rel_out_hbm,
          bufa_v, bufb_v, pieceb_v, csem0, csem1, psem0, psem1,
          tail_v, relbuf_v, piece_v, rels_v, mli_v, mlo_v,
          wli_v, wlo_v, stag0_v, stag1_v, sem0, sem1):
        wid = lax.axis_index("s") * nc + lax.axis_index("c")
        iota16 = lax.iota(jnp.int32, 16)
        stags = (stag0_v, stag1_v)
        sems = (sem0, sem1)
        bufs = (bufa_v, bufb_v)
        csems = (csem0, csem1)
        pieces = (piece_v, pieceb_v)
        psems = (psem0, psem1)

        def ring_wait(b):
            pltpu.make_async_copy(
                stags[b], out_hbm.at[pl.ds(NI, 16)], sems[b]).wait()

        _rel_scope = jax.named_scope("ph_rel"); _rel_scope.__enter__()
        pltpu.sync_copy(rel_hbm, relbuf_v)
        pltpu.sync_copy(rels_hbm.at[pl.ds(wid * BPW, BPW)], rels_v)

        def rel_grp(g, _):
            for b in (0, 1):
                @pl.when((g & 1) == b)
                def _b():
                    @pl.when(g >= 2)
                    def _w():
                        ring_wait(b)

                    rvec = rels_v[pl.ds(g * 16, 16)]
                    for c0 in range(DIM):
                        cvec = jnp.full((16,), c0, jnp.int32)
                        vals = plsc.load_gather(relbuf_v, [cvec, rvec])
                        plsc.store_scatter(stags[b], [iota16, cvec], vals)
                    pltpu.async_copy(
                        stags[b],
                        rel_out_hbm.at[pl.ds(wid * BPW + g * 16, 16)],
                        sems[b])

            return 0

        lax.fori_loop(0, BPW // 16, rel_grp, 0)
        ring_wait(0)
        ring_wait(1)
        _rel_scope.__exit__(None, None, None)

        def scan(rnd):
            lo = rnd * CAP

            def piece_issue(p, b):
                pltpu.async_copy(
                    idx_hbm.at[pl.ds(p * PIECE, PIECE)], pieces[b], psems[b])

            def piece_wait(b):
                pltpu.make_async_copy(
                    idx_hbm.at[pl.ds(0, PIECE)], pieces[b], psems[b]).wait()

            piece_issue(0, 0)
            piece_issue(1, 1)

            def scan_piece(p, cnt):
                par0 = (p & 1) == 0

                for b in (0, 1):
                    @pl.when((p & 1) == b)
                    def _w():
                        piece_wait(b)

                def scan_vec(i, cnt):
                    v0 = pieces[0][pl.ds(i * 16, 16)]
                    v1 = pieces[1][pl.ds(i * 16, 16)]
                    vec = jnp.where(par0, v0, v1)
                    mine = ((vec >> WSH) & 31) == wid
                    mi = mine.astype(jnp.int32)
                    rank = cnt + lax.cumsum(mi, axis=0) - mi
                    keep = mine & (rank >= lo) & (rank < lo + CAP)
                    pos = jnp.clip(rank - lo, 0, CAP - 1)
                    org = p * PIECE + i * 16 + iota16
                    plsc.store_scatter(mli_v, [pos], vec, mask=keep)
                    plsc.store_scatter(mlo_v, [pos], org, mask=keep)
                    return cnt + plsc.all_reduce_population_count(mine)

                cnt = lax.fori_loop(0, PIECE // 16, scan_vec, cnt)

                for b in (0, 1):
                    @pl.when(((p & 1) == b) & (p + 2 < NPIECE))
                    def _i():
                        piece_issue(p + 2, b)

                return cnt

            cnt = lax.fori_loop(0, NPIECE, scan_piece,
                                jnp.zeros((16,), jnp.int32))
            return jnp.max(cnt)

        def compact(ch, mr):
            def cvec_it(ki, c2):
                vec = mli_v[pl.ds(ki * 16, 16)]
                org = mlo_v[pl.ds(ki * 16, 16)]
                m = ((ki * 16 + iota16) < mr) & ((vec >> WSH) == ch)
                mi = m.astype(jnp.int32)
                pos = c2 + lax.cumsum(mi, axis=0) - mi
                plsc.store_scatter(wli_v, [pos], vec, mask=m)
                plsc.store_scatter(wlo_v, [pos], org, mask=m)
                return c2 + plsc.all_reduce_population_count(m)

            c2 = lax.fori_loop(0, (mr + 15) // 16, cvec_it,
                               jnp.zeros((16,), jnp.int32))
            return jnp.max(c2)

        def extract(src_v, width, base, c2):
            def grp(g, _):
                for b in (0, 1):
                    @pl.when((g & 1) == b)
                    def _b():
                        @pl.when(g >= 2)
                        def _w():
                            ring_wait(b)

                        lanes0 = wli_v[pl.ds(g * 16, 16)]
                        valid = (g * 16 + iota16) < c2
                        lanes = jnp.clip(jnp.where(valid, lanes0 - base, 0),
                                         0, width - 1)
                        for c0 in range(DIM):
                            cvec = jnp.full((16,), c0, jnp.int32)
                            vals = plsc.load_gather(src_v, [cvec, lanes])
                            plsc.store_scatter(stags[b], [iota16, cvec], vals)
                        org = jnp.where(valid, wlo_v[pl.ds(g * 16, 16)],
                                        NI + iota16)
                        pltpu.async_copy(stags[b], out_hbm.at[org], sems[b])

                return 0

            ngrp = (c2 + 15) // 16
            lax.fori_loop(0, ngrp, grp, 0)

            @pl.when(ngrp >= 1)
            def _d0():
                ring_wait(0)

            @pl.when(ngrp >= 2)
            def _d1():
                ring_wait(1)

        def chunk_issue(j, b):
            ch = wid + 32 * j
            pltpu.async_copy(
                tab_hbm.at[:, pl.ds(ch * W, W)], bufs[b], csems[b])

        def chunk_wait(b):
            pltpu.make_async_copy(
                tab_hbm.at[:, pl.ds(0, W)], bufs[b], csems[b]).wait()

        def round_body(carry):
            rnd, _ = carry
            with jax.named_scope("ph_scan"):
                cnt = scan(rnd)
            mr = jnp.clip(cnt - rnd * CAP, 0, CAP)
            nfm = (NFULL - 1 - wid) // 32 + 1

            chunk_issue(0, 0)

            @pl.when(nfm >= 2)
            def _p1():
                chunk_issue(1, 1)

            def do_chunk(j, _):
                for b in (0, 1):
                    @pl.when((j & 1) == b)
                    def _b():
                        chunk_wait(b)
                        ch = wid + 32 * j
                        extract(bufs[b], W, ch * W, compact(ch, mr))

                        @pl.when(j + 2 < nfm)
                        def _pf():
                            chunk_issue(j + 2, b)

                return 0

            _ch_scope = jax.named_scope("ph_chunks"); _ch_scope.__enter__()
            lax.fori_loop(0, nfm, do_chunk, 0)
            _ch_scope.__exit__(None, None, None)

            @pl.when(wid == (TAIL_ID & 31))
            def _tail():
                pltpu.sync_copy(
                    tab_hbm.at[:, pl.ds(NFULL * W, TAILW)], tail_v)
                extract(tail_v, TAILW, NFULL * W, compact(TAIL_ID, mr))

            return rnd + 1, cnt

        lax.while_loop(lambda c: c[0] * CAP < c[1], round_body,
                       (jnp.int32(0), jnp.int32(1)))

    return k(tab_t, rel_t, all_idx, rels)


def _tc_loss(h_rows, t_rows, c_rows, r_rows):
    blk = 2048
    nblk = B // blk

    def body(h_ref, t_ref, c_ref, r_ref, o_ref, acc_ref):
        step = pl.program_id(0)

        @pl.when(step == 0)
        def _init():
            acc_ref[0, 0] = 0.0

        h = h_ref[...][:, :DIM]
        t = t_ref[...][:, :DIM]
        c = c_ref[...][:, :DIM]
        r = r_ref[...][:, :DIM]
        h = h / jnp.sqrt(jnp.sum(h * h, axis=1, keepdims=True))
        t = t / jnp.sqrt(jnp.sum(t * t, axis=1, keepdims=True))
        c = c / jnp.sqrt(jnp.sum(c * c, axis=1, keepdims=True))
        dp = h + r - t
        dn = h + r - c
        pos_dist = jnp.sqrt(jnp.sum(dp * dp, axis=1) + EPS)
        neg_dist = jnp.sqrt(jnp.sum(dn * dn, axis=1) + EPS)
        terms = jnp.maximum(0.0, MARGIN + pos_dist - neg_dist)
        acc_ref[0, 0] += jnp.sum(terms)

        @pl.when(step == nblk - 1)
        def _fini():
            o_ref[0, 0] = acc_ref[0, 0] * (1.0 / B)

    row_spec = pl.BlockSpec((blk, 128), lambda i: (i, 0))
    out = pl.pallas_call(
        body,
        grid=(nblk,),
        in_specs=[row_spec] * 4,
        out_specs=pl.BlockSpec(memory_space=pltpu.SMEM),
        out_shape=jax.ShapeDtypeStruct((1, 1), jnp.float32),
        scratch_shapes=[pltpu.SMEM((1, 1), jnp.float32)],
    )(h_rows, t_rows, c_rows, r_rows)
    return out[0, 0]


def kernel(triplets, corrupted_triplets, entity_emb, relation_emb):
    heads = triplets[:, 0].astype(jnp.int32)
    rels = triplets[:, 1].astype(jnp.int32)
    tails = triplets[:, 2].astype(jnp.int32)
    ctails = corrupted_triplets[:, 2].astype(jnp.int32)
    all_idx = jnp.concatenate([heads, tails, ctails])
    ent_rows, rel_rows = _sc_gather(
        entity_emb.T, relation_emb.T, all_idx, rels)
    return _tc_loss(ent_rows[:B], ent_rows[B:2 * B], ent_rows[2 * B:3 * B],
                    rel_rows)
